# baseline (speedup 1.0000x reference)
import jax
import jax.numpy as jnp
from jax import lax
from jax.experimental import pallas as pl
from jax.experimental.pallas import tpu as pltpu
from jax.experimental.pallas import tpu_sc as plsc

_N = 524288
_LANES = 16

_info = plsc.get_sparse_core_info()
_NC = _info.num_cores
_NS = _info.num_subcores
_NW = _NC * _NS

_K = 131072
_CHUNK = _K // _NW
_NBUF = 2
_SUB = _CHUNK // _NBUF

_ROWS = _N // 128
_BLK = 512
_KBLKS = _K // 128 // _BLK
_NBLKS = _ROWS // _BLK


def _sc_clamp_body(x_hbm, vals_hbm, out_hbm, vals_v, buf_v, in_sem, out_sem):
    wid = lax.axis_index("s") * _NC + lax.axis_index("c")
    base = wid * _CHUNK

    in_copies = [
        pltpu.async_copy(
            x_hbm.at[pl.ds(base + j * _SUB, _SUB)],
            buf_v.at[pl.ds(j * _SUB, _SUB)],
            in_sem,
        )
        for j in range(_NBUF)
    ]

    pltpu.sync_copy(vals_hbm, vals_v)
    head = vals_v[pl.ds(0, _LANES)]
    tail = vals_v[pl.ds(64 - _LANES, _LANES)]
    lo_vec = jnp.full((_LANES,), head[0], jnp.float32)
    hi_vec = jnp.full((_LANES,), tail[_LANES - 1], jnp.float32)

    out_copies = []
    for j in range(_NBUF):
        in_copies[j].wait()

        @plsc.parallel_loop(j * _SUB, (j + 1) * _SUB, _LANES, unroll=8)
        def _(i):
            sl = pl.ds(i, _LANES)
            buf_v[sl] = jnp.minimum(jnp.maximum(buf_v[sl], lo_vec), hi_vec)

        out_copies.append(
            pltpu.async_copy(
                buf_v.at[pl.ds(j * _SUB, _SUB)],
                out_hbm.at[pl.ds(base + j * _SUB, _SUB)],
                out_sem,
            )
        )

    for c in out_copies:
        c.wait()


def _tc_merge_body(vals_s, x_ref, sc_ref, o_ref):
    j = pl.program_id(0)
    lo = vals_s[0]
    hi = vals_s[63]

    @pl.when(j < _KBLKS)
    def _():
        o_ref[...] = sc_ref[...]

    @pl.when(j >= _KBLKS)
    def _():
        o_ref[...] = jnp.minimum(jnp.maximum(x_ref[...], lo), hi)


@jax.jit
def kernel(tensor_input, values):
    x = tensor_input.reshape(_N)

    sc_out = pl.kernel(
        _sc_clamp_body,
        out_type=jax.ShapeDtypeStruct((_K,), jnp.float32),
        mesh=plsc.VectorSubcoreMesh(core_axis_name="c", subcore_axis_name="s"),
        scratch_types=[
            pltpu.VMEM((64,), jnp.float32),
            pltpu.VMEM((_CHUNK,), jnp.float32),
            pltpu.SemaphoreType.DMA,
            pltpu.SemaphoreType.DMA,
        ],
    )(x, values)

    merged = pl.pallas_call(
        _tc_merge_body,
        grid=(_NBLKS,),
        in_specs=[
            pl.BlockSpec(memory_space=pltpu.SMEM),
            pl.BlockSpec((_BLK, 128), lambda j: (j, 0)),
            pl.BlockSpec((_BLK, 128), lambda j: (jnp.minimum(j, _KBLKS - 1), 0)),
        ],
        out_specs=pl.BlockSpec((_BLK, 128), lambda j: (j, 0)),
        out_shape=jax.ShapeDtypeStruct((_ROWS, 128), jnp.float32),
    )(values, x.reshape(_ROWS, 128), sc_out.reshape(_K // 128, 128))

    return merged.reshape(_N, 1)


# speedup vs baseline: 1.0935x; 1.0935x over previous
import jax
import jax.numpy as jnp
from jax import lax
from jax.experimental import pallas as pl
from jax.experimental.pallas import tpu as pltpu
from jax.experimental.pallas import tpu_sc as plsc

_N = 524288
_LANES = 16

_info = plsc.get_sparse_core_info()
_NC = _info.num_cores
_NS = _info.num_subcores
_NW = _NC * _NS

_K = 131072
_CHUNK = _K // _NW
_NBUF = 2
_SUB = _CHUNK // _NBUF

_ROWS = _N // 128
_BLK = 512
_KBLKS = _K // 128 // _BLK
_NBLKS = _ROWS // _BLK


def _tc_body(vals_s, x_ref, o_ref):
    j = pl.program_id(0)

    @pl.when(j < _KBLKS)
    def _():
        o_ref[...] = x_ref[...]

    @pl.when(j >= _KBLKS)
    def _():
        o_ref[...] = jnp.minimum(
            jnp.maximum(x_ref[...], vals_s[0]), vals_s[63]
        )


def _sc_fill_body(buf_hbm, vals_hbm, vals_v, tile_v, in_sem, out_sem):
    wid = lax.axis_index("s") * _NC + lax.axis_index("c")
    base = wid * _CHUNK

    in_copies = [
        pltpu.async_copy(
            buf_hbm.at[pl.ds(base + j * _SUB, _SUB)],
            tile_v.at[pl.ds(j * _SUB, _SUB)],
            in_sem,
        )
        for j in range(_NBUF)
    ]

    pltpu.sync_copy(vals_hbm, vals_v)
    head = vals_v[pl.ds(0, _LANES)]
    tail = vals_v[pl.ds(64 - _LANES, _LANES)]
    lo_vec = jnp.full((_LANES,), head[0], jnp.float32)
    hi_vec = jnp.full((_LANES,), tail[_LANES - 1], jnp.float32)

    out_copies = []
    for j in range(_NBUF):
        in_copies[j].wait()

        @plsc.parallel_loop(j * _SUB, (j + 1) * _SUB, _LANES, unroll=8)
        def _(i):
            sl = pl.ds(i, _LANES)
            tile_v[sl] = jnp.minimum(jnp.maximum(tile_v[sl], lo_vec), hi_vec)

        out_copies.append(
            pltpu.async_copy(
                tile_v.at[pl.ds(j * _SUB, _SUB)],
                buf_hbm.at[pl.ds(base + j * _SUB, _SUB)],
                out_sem,
            )
        )

    for c in out_copies:
        c.wait()


_sc_fill = pl.kernel(
    _sc_fill_body,
    out_type=(),
    mesh=plsc.VectorSubcoreMesh(core_axis_name="c", subcore_axis_name="s"),
    scratch_types=[
        pltpu.VMEM((64,), jnp.float32),
        pltpu.VMEM((_CHUNK,), jnp.float32),
        pltpu.SemaphoreType.DMA,
        pltpu.SemaphoreType.DMA,
    ],
)


@jax.jit
def kernel(tensor_input, values):
    x = tensor_input.reshape(_ROWS, 128)

    staged = pl.pallas_call(
        _tc_body,
        grid=(_NBLKS,),
        in_specs=[
            pl.BlockSpec(memory_space=pltpu.SMEM),
            pl.BlockSpec((_BLK, 128), lambda j: (j, 0)),
        ],
        out_specs=pl.BlockSpec((_BLK, 128), lambda j: (j, 0)),
        out_shape=jax.ShapeDtypeStruct((_ROWS, 128), jnp.float32),
    )(values, x)

    buf = jax.new_ref(staged.reshape(_N))
    _sc_fill(buf, values)
    return buf[...].reshape(_N, 1)


# speedup vs baseline: 1.2033x; 1.1005x over previous
import jax
import jax.numpy as jnp
from jax import lax
from jax.experimental import pallas as pl
from jax.experimental.pallas import tpu as pltpu
from jax.experimental.pallas import tpu_sc as plsc

_N = 524288
_LANES = 16

_info = plsc.get_sparse_core_info()
_NC = _info.num_cores
_NS = _info.num_subcores
_NW = _NC * _NS
_CHUNK = _N // _NW
_NBUF = 2
_SUB = _CHUNK // _NBUF


def _clamp_body(x_hbm, vals_hbm, out_hbm, vals_v, buf_v, in_sem, out_sem):
    wid = lax.axis_index("s") * _NC + lax.axis_index("c")
    base = wid * _CHUNK

    in_copies = [
        pltpu.async_copy(
            x_hbm.at[pl.ds(base + j * _SUB, _SUB)],
            buf_v.at[pl.ds(j * _SUB, _SUB)],
            in_sem,
        )
        for j in range(_NBUF)
    ]

    pltpu.sync_copy(vals_hbm, vals_v)
    head = vals_v[pl.ds(0, _LANES)]
    tail = vals_v[pl.ds(64 - _LANES, _LANES)]
    lo_vec = jnp.full((_LANES,), head[0], jnp.float32)
    hi_vec = jnp.full((_LANES,), tail[_LANES - 1], jnp.float32)

    out_copies = []
    for j in range(_NBUF):
        in_copies[j].wait()

        @plsc.parallel_loop(j * _SUB, (j + 1) * _SUB, _LANES, unroll=8)
        def _(i):
            sl = pl.ds(i, _LANES)
            buf_v[sl] = jnp.minimum(jnp.maximum(buf_v[sl], lo_vec), hi_vec)

        out_copies.append(
            pltpu.async_copy(
                buf_v.at[pl.ds(j * _SUB, _SUB)],
                out_hbm.at[pl.ds(base + j * _SUB, _SUB)],
                out_sem,
            )
        )

    for c in out_copies:
        c.wait()


@jax.jit
def kernel(tensor_input, values):
    x = tensor_input.reshape(_N)
    out = pl.kernel(
        _clamp_body,
        out_type=jax.ShapeDtypeStruct((_N,), jnp.float32),
        mesh=plsc.VectorSubcoreMesh(core_axis_name="c", subcore_axis_name="s"),
        scratch_types=[
            pltpu.VMEM((64,), jnp.float32),
            pltpu.VMEM((_CHUNK,), jnp.float32),
            pltpu.SemaphoreType.DMA,
            pltpu.SemaphoreType.DMA,
        ],
    )(x, values)
    return out.reshape(_N, 1)
